# baseline (speedup 1.0000x reference)
import functools

import jax
import jax.numpy as jnp
from jax import lax
from jax.experimental import pallas as pl
from jax.experimental.pallas import tpu as pltpu
from jax.experimental.pallas import tpu_sc as plsc

N_NODES = 10000
HID = 128
K = 8
D = 16
ROUTIT = 7
NUM_LAYERS = 4

NC = 2
NS = 16
WORKERS = NC * NS
B = 32
GPB = B // 16
NPAD = 10240
ROWS_PER_TILE = NPAD // NS
SCH = 32
BLK_R = 256


def _group_mat():
    r = lax.broadcasted_iota(jnp.int32, (HID, HID), 0) // D
    c = lax.broadcasted_iota(jnp.int32, (HID, HID), 1) // D
    return (r == c).astype(jnp.float32)


def _inv_norm(s):
    return jnp.minimum(lax.rsqrt(s), 1e12)


def _init_body(x_ref, w_ref, b_ref, o_ref):
    i = pl.program_id(0)
    z = jnp.dot(x_ref[...], w_ref[...], preferred_element_type=jnp.float32)
    z = jnp.maximum(z + b_ref[...], 0.0)
    row = i * BLK_R + lax.broadcasted_iota(jnp.int32, (BLK_R, HID), 0)
    z = jnp.where(row < N_NODES, z, 0.0)
    s = jnp.dot(z * z, _group_mat(), preferred_element_type=jnp.float32)
    o_ref[...] = z * _inv_norm(s)


_init_call = pl.pallas_call(
    _init_body,
    grid=(NPAD // BLK_R,),
    in_specs=[
        pl.BlockSpec((BLK_R, HID), lambda i: (i, 0)),
        pl.BlockSpec((HID, HID), lambda i: (0, 0)),
        pl.BlockSpec((1, HID), lambda i: (0, 0)),
    ],
    out_specs=pl.BlockSpec((BLK_R, HID), lambda i: (i, 0)),
    out_shape=jax.ShapeDtypeStruct((NPAD, HID), jnp.float32),
)


def _norm_body(mode, agg_ref, x_ref, o_ref):
    g = _group_mat()
    t = agg_ref[0] + agg_ref[1] + x_ref[...]
    s = jnp.dot(t * t, g, preferred_element_type=jnp.float32)
    u = t * _inv_norm(s)
    if mode == "mid":
        o_ref[...] = u
    elif mode == "final":
        o_ref[...] = jnp.maximum(u, 0.0)
    else:
        r = jnp.maximum(u, 0.0)
        s2 = jnp.dot(r * r, g, preferred_element_type=jnp.float32)
        o_ref[...] = r * _inv_norm(s2)


def _make_norm(mode):
    return pl.pallas_call(
        functools.partial(_norm_body, mode),
        grid=(NPAD // BLK_R,),
        in_specs=[
            pl.BlockSpec((NC, BLK_R, HID), lambda i: (0, i, 0)),
            pl.BlockSpec((BLK_R, HID), lambda i: (i, 0)),
        ],
        out_specs=pl.BlockSpec((BLK_R, HID), lambda i: (i, 0)),
        out_shape=jax.ShapeDtypeStruct((NPAD, HID), jnp.float32),
    )


_norm_mid = _make_norm("mid")
_norm_end = _make_norm("end")
_norm_final = _make_norm("final")


SB = 8


def _make_edge_kernel(nsb):
    mesh = plsc.VectorSubcoreMesh(core_axis_name="c", subcore_axis_name="s")

    @functools.partial(
        pl.kernel,
        out_type=jax.ShapeDtypeStruct((NC, NPAD, HID), jnp.float32),
        mesh=mesh,
        compiler_params=pltpu.CompilerParams(needs_layout_passes=False),
        scratch_types=[
            pltpu.VMEM((SB, B), jnp.int32),
            pltpu.VMEM((SB, B), jnp.int32),
            pltpu.VMEM((B, HID), jnp.float32),
            pltpu.VMEM((B, HID), jnp.float32),
            pltpu.VMEM((B, HID), jnp.float32),
            pltpu.VMEM((B, HID), jnp.float32),
            pltpu.VMEM((B, HID), jnp.float32),
            pltpu.VMEM((D * K, 16), jnp.int32),
            pltpu.VMEM_SHARED((NPAD, HID), jnp.float32),
            pltpu.SemaphoreType.DMA,
            pltpu.SemaphoreType.DMA,
            pltpu.SemaphoreType.DMA,
            pltpu.SemaphoreType.DMA,
            pltpu.SemaphoreType.DMA,
        ],
    )
    def edge_kernel(x_ref, u_ref, src_ref, dst_ref, out_ref,
                    srcv, dstv, z0, z1, u0, u1, mbuf, coltab, aggsh,
                    sz0, sz1, su0, su1, ssc):
        c = lax.axis_index("c")
        s = lax.axis_index("s")
        wid = c * NS + s
        base = s * ROWS_PER_TILE

        def zb(i, carry):
            for jj in range(HID // 16):
                mbuf[i, pl.ds(jj * 16, 16)] = jnp.zeros((16,), jnp.float32)
            return carry
        lax.fori_loop(0, B, zb, 0)
        for i in range(ROWS_PER_TILE // SCH):
            pltpu.sync_copy(mbuf.at[pl.ds(0, SCH)],
                            aggsh.at[pl.ds(base + i * SCH, SCH)])
        plsc.subcore_barrier()

        iota16 = lax.iota(jnp.int32, 16)
        zbufs = (z0, z1)
        ubufs = (u0, u1)
        szs = (sz0, sz1)
        sus = (su0, su1)

        def issue_gather(b, par):
            pltpu.async_copy(x_ref.at[srcv.at[b]], zbufs[par], szs[par])
            pltpu.async_copy(u_ref.at[dstv.at[b]], ubufs[par], sus[par])

        def wait_gather(b, par):
            pltpu.make_async_copy(x_ref.at[srcv.at[b]], zbufs[par], szs[par]).wait()
            pltpu.make_async_copy(u_ref.at[dstv.at[b]], ubufs[par], sus[par]).wait()

        def wait_scatter(b):
            pltpu.make_async_copy(mbuf, aggsh.at[dstv.at[b]], ssc).wait()

        half0 = iota16 >> 1
        for t in range(D):
            jj = ((iota16 + t) & 1) * 8 + ((half0 + (t >> 1)) & 7)
            for d in range(K):
                coltab[t * K + d, :] = (((half0 + d) & 7) * 16) + jj

        def compute_block(zb, ub):
            def gbody(g, gcarry):
                rows = g * 16 + iota16

                def tbodyP(t, accs):
                    out = []
                    for d in range(K):
                        cols = coltab[t * K + d]
                        zv = plsc.load_gather(zb, [rows, cols])
                        uv = plsc.load_gather(ub, [rows, cols])
                        out.append(accs[d] + zv * uv)
                    return tuple(out)
                accs = lax.fori_loop(
                    0, D, tbodyP,
                    tuple(jnp.zeros((16,), jnp.float32) for _ in range(K)))
                m = accs[0]
                for d in range(1, K):
                    m = jnp.maximum(m, accs[d])
                es = [jnp.exp(a - m) for a in accs]
                tot = es[0]
                for d in range(1, K):
                    tot = tot + es[d]
                inv = 1.0 / tot
                ws = [e * inv for e in es]

                def tbodyM(t, mcarry):
                    for d in range(K):
                        cols = coltab[t * K + d]
                        zv = plsc.load_gather(zb, [rows, cols])
                        plsc.store_scatter(mbuf, [rows, cols], zv * ws[d])
                    return mcarry
                lax.fori_loop(0, D, tbodyM, 0)
                return gcarry
            lax.fori_loop(0, GPB, gbody, 0)

        def sbody(sb, carry):
            @pl.when(sb > 0)
            def _():
                wait_scatter(0)
            pltpu.sync_copy(src_ref.at[wid, pl.ds(sb * SB, SB)], srcv)
            pltpu.sync_copy(dst_ref.at[wid, pl.ds(sb * SB, SB)], dstv)
            issue_gather(0, 0)

            def pbody2(p, pcarry):
                bA = 2 * p
                issue_gather(bA + 1, 1)
                wait_gather(bA, 0)

                @pl.when(p > 0)
                def _():
                    wait_scatter(bA)
                compute_block(z0, u0)
                pltpu.async_copy(mbuf, aggsh.at[dstv.at[bA]], ssc, add=True)

                @pl.when(p < SB // 2 - 1)
                def _():
                    issue_gather(bA + 2, 0)
                wait_gather(bA + 1, 1)
                wait_scatter(bA + 1)
                compute_block(z1, u1)
                pltpu.async_copy(mbuf, aggsh.at[dstv.at[bA + 1]], ssc, add=True)
                return pcarry
            lax.fori_loop(0, SB // 2, pbody2, 0)
            return carry
        lax.fori_loop(0, nsb, sbody, 0)

        wait_scatter(0)
        plsc.subcore_barrier()
        for i in range(ROWS_PER_TILE // SCH):
            pltpu.sync_copy(aggsh.at[pl.ds(base + i * SCH, SCH)],
                            mbuf.at[pl.ds(0, SCH)])
            pltpu.sync_copy(mbuf.at[pl.ds(0, SCH)],
                            out_ref.at[c, pl.ds(base + i * SCH, SCH)])

    return edge_kernel


def kernel(X, edges, W_init, b_init):
    n, _ = X.shape
    e = edges.shape[1]
    chunk = WORKERS * B * SB
    epad = -(-e // chunk) * chunk
    nsb = epad // chunk
    nblk = nsb * SB

    Xp = jnp.pad(X, ((0, NPAD - n), (0, 0)))
    src = jnp.pad(edges[0], (0, epad - e), constant_values=NPAD - 1)
    dst = jnp.pad(edges[1], (0, epad - e), constant_values=NPAD - 1)
    src3 = src.reshape(WORKERS, nblk, B)
    dst3 = dst.reshape(WORKERS, nblk, B)

    edge_call = _make_edge_kernel(nsb)

    x = _init_call(Xp, W_init, b_init.reshape(1, HID))
    out = None
    for layer in range(NUM_LAYERS):
        u = x
        for it in range(ROUTIT):
            agg = edge_call(x, u, src3, dst3)
            if it < ROUTIT - 1:
                u = _norm_mid(agg, x)
            elif layer < NUM_LAYERS - 1:
                x = _norm_end(agg, x)
            else:
                out = _norm_final(agg, x)
    return out[:n]


# speedup vs baseline: 1.8252x; 1.8252x over previous
import functools

import jax
import jax.numpy as jnp
from jax import lax
from jax.experimental import pallas as pl
from jax.experimental.pallas import tpu as pltpu
from jax.experimental.pallas import tpu_sc as plsc

N_NODES = 10000
HID = 128
K = 8
D = 16
ROUTIT = 7
NUM_LAYERS = 4

NC = 2
NS = 16
WORKERS = NC * NS
B = 32
GPB = B // 16
NPAD = 10240
ROWS_PER_TILE = NPAD // NS
SCH = 32
BLK_R = 256


def _group_mat():
    r = lax.broadcasted_iota(jnp.int32, (HID, HID), 0) // D
    c = lax.broadcasted_iota(jnp.int32, (HID, HID), 1) // D
    return (r == c).astype(jnp.float32)


def _inv_norm(s):
    return jnp.minimum(lax.rsqrt(s), 1e12)


def _init_body(x_ref, w_ref, b_ref, o_ref):
    i = pl.program_id(0)
    z = jnp.dot(x_ref[...], w_ref[...], preferred_element_type=jnp.float32)
    z = jnp.maximum(z + b_ref[...], 0.0)
    row = i * BLK_R + lax.broadcasted_iota(jnp.int32, (BLK_R, HID), 0)
    z = jnp.where(row < N_NODES, z, 0.0)
    s = jnp.dot(z * z, _group_mat(), preferred_element_type=jnp.float32)
    o_ref[...] = z * _inv_norm(s)


_init_call = pl.pallas_call(
    _init_body,
    grid=(NPAD // BLK_R,),
    in_specs=[
        pl.BlockSpec((BLK_R, HID), lambda i: (i, 0)),
        pl.BlockSpec((HID, HID), lambda i: (0, 0)),
        pl.BlockSpec((1, HID), lambda i: (0, 0)),
    ],
    out_specs=pl.BlockSpec((BLK_R, HID), lambda i: (i, 0)),
    out_shape=jax.ShapeDtypeStruct((NPAD, HID), jnp.float32),
)


def _norm_body(mode, agg_ref, x_ref, o_ref):
    g = _group_mat()
    t = agg_ref[0] + agg_ref[1] + x_ref[...]
    s = jnp.dot(t * t, g, preferred_element_type=jnp.float32)
    u = t * _inv_norm(s)
    if mode == "mid":
        o_ref[...] = u
    elif mode == "final":
        o_ref[...] = jnp.maximum(u, 0.0)
    else:
        r = jnp.maximum(u, 0.0)
        s2 = jnp.dot(r * r, g, preferred_element_type=jnp.float32)
        o_ref[...] = r * _inv_norm(s2)


def _make_norm(mode):
    return pl.pallas_call(
        functools.partial(_norm_body, mode),
        grid=(NPAD // BLK_R,),
        in_specs=[
            pl.BlockSpec((NC, BLK_R, HID), lambda i: (0, i, 0)),
            pl.BlockSpec((BLK_R, HID), lambda i: (i, 0)),
        ],
        out_specs=pl.BlockSpec((BLK_R, HID), lambda i: (i, 0)),
        out_shape=jax.ShapeDtypeStruct((NPAD, HID), jnp.float32),
    )


_norm_mid = _make_norm("mid")
_norm_end = _make_norm("end")
_norm_final = _make_norm("final")


SB = 8


def _make_edge_kernel(nsb):
    mesh = plsc.VectorSubcoreMesh(core_axis_name="c", subcore_axis_name="s")

    @functools.partial(
        pl.kernel,
        out_type=jax.ShapeDtypeStruct((NC, NPAD, HID), jnp.float32),
        mesh=mesh,
        compiler_params=pltpu.CompilerParams(needs_layout_passes=False),
        scratch_types=[
            pltpu.VMEM((SB, B), jnp.int32),
            pltpu.VMEM((SB, B), jnp.int32),
            pltpu.VMEM((B, HID), jnp.float32),
            pltpu.VMEM((B, HID), jnp.float32),
            pltpu.VMEM((B, HID), jnp.float32),
            pltpu.VMEM((B, HID), jnp.float32),
            pltpu.VMEM((B, HID), jnp.float32),
            pltpu.VMEM_SHARED((NPAD, HID), jnp.float32),
            pltpu.SemaphoreType.DMA,
            pltpu.SemaphoreType.DMA,
            pltpu.SemaphoreType.DMA,
            pltpu.SemaphoreType.DMA,
            pltpu.SemaphoreType.DMA,
        ],
    )
    def edge_kernel(x_ref, u_ref, src_ref, dst_ref, out_ref,
                    srcv, dstv, z0, z1, u0, u1, mbuf, aggsh,
                    sz0, sz1, su0, su1, ssc):
        c = lax.axis_index("c")
        s = lax.axis_index("s")
        wid = c * NS + s
        base = s * ROWS_PER_TILE

        def zb(i, carry):
            for jj in range(HID // 16):
                mbuf[i, pl.ds(jj * 16, 16)] = jnp.zeros((16,), jnp.float32)
            return carry
        lax.fori_loop(0, B, zb, 0)
        for i in range(ROWS_PER_TILE // SCH):
            pltpu.sync_copy(mbuf.at[pl.ds(0, SCH)],
                            aggsh.at[pl.ds(base + i * SCH, SCH)])
        plsc.subcore_barrier()

        iota16 = lax.iota(jnp.int32, 16)
        zbufs = (z0, z1)
        ubufs = (u0, u1)
        szs = (sz0, sz1)
        sus = (su0, su1)

        def issue_gather(b, par):
            pltpu.async_copy(x_ref.at[srcv.at[b]], zbufs[par], szs[par])
            pltpu.async_copy(u_ref.at[dstv.at[b]], ubufs[par], sus[par])

        def wait_gather(b, par):
            pltpu.make_async_copy(x_ref.at[srcv.at[b]], zbufs[par], szs[par]).wait()
            pltpu.make_async_copy(u_ref.at[dstv.at[b]], ubufs[par], sus[par]).wait()

        def wait_scatter(b):
            pltpu.make_async_copy(mbuf, aggsh.at[dstv.at[b]], ssc).wait()

        kmasks = [iota16 == k for k in range(K)]
        kidx = [jnp.full((16,), k, jnp.int32) for k in range(K)]
        neg = jnp.full((16,), -1e30, jnp.float32)

        def compute_block(zb, ub):
            @plsc.parallel_loop(0, B, 1, unroll=4)
            def ebody(e):
                zrows = [zb[e, pl.ds(k * D, D)] for k in range(K)]
                pv = neg
                for k in range(K):
                    uc = ub[e, pl.ds(k * D, D)]
                    pk = jnp.sum(zrows[k] * uc)
                    pv = jnp.where(kmasks[k], pk, pv)
                m = jnp.max(pv)
                ev = jnp.exp(pv - m)
                wv = ev / jnp.sum(ev)
                for k in range(K):
                    wbk = wv.at[kidx[k]].get(mode="promise_in_bounds")
                    mbuf[e, pl.ds(k * D, D)] = zrows[k] * wbk

        def sbody(sb, carry):
            @pl.when(sb > 0)
            def _():
                wait_scatter(0)
            pltpu.sync_copy(src_ref.at[wid, pl.ds(sb * SB, SB)], srcv)
            pltpu.sync_copy(dst_ref.at[wid, pl.ds(sb * SB, SB)], dstv)
            issue_gather(0, 0)

            def pbody2(p, pcarry):
                bA = 2 * p
                issue_gather(bA + 1, 1)
                wait_gather(bA, 0)

                @pl.when(p > 0)
                def _():
                    wait_scatter(bA)
                compute_block(z0, u0)
                pltpu.async_copy(mbuf, aggsh.at[dstv.at[bA]], ssc, add=True)

                @pl.when(p < SB // 2 - 1)
                def _():
                    issue_gather(bA + 2, 0)
                wait_gather(bA + 1, 1)
                wait_scatter(bA + 1)
                compute_block(z1, u1)
                pltpu.async_copy(mbuf, aggsh.at[dstv.at[bA + 1]], ssc, add=True)
                return pcarry
            lax.fori_loop(0, SB // 2, pbody2, 0)
            return carry
        lax.fori_loop(0, nsb, sbody, 0)

        wait_scatter(0)
        plsc.subcore_barrier()
        for i in range(ROWS_PER_TILE // SCH):
            pltpu.sync_copy(aggsh.at[pl.ds(base + i * SCH, SCH)],
                            mbuf.at[pl.ds(0, SCH)])
            pltpu.sync_copy(mbuf.at[pl.ds(0, SCH)],
                            out_ref.at[c, pl.ds(base + i * SCH, SCH)])

    return edge_kernel


def kernel(X, edges, W_init, b_init):
    n, _ = X.shape
    e = edges.shape[1]
    chunk = WORKERS * B * SB
    epad = -(-e // chunk) * chunk
    nsb = epad // chunk
    nblk = nsb * SB

    Xp = jnp.pad(X, ((0, NPAD - n), (0, 0)))
    src = jnp.pad(edges[0], (0, epad - e), constant_values=NPAD - 1)
    dst = jnp.pad(edges[1], (0, epad - e), constant_values=NPAD - 1)
    src3 = src.reshape(WORKERS, nblk, B)
    dst3 = dst.reshape(WORKERS, nblk, B)

    edge_call = _make_edge_kernel(nsb)

    x = _init_call(Xp, W_init, b_init.reshape(1, HID))
    out = None
    for layer in range(NUM_LAYERS):
        u = x
        for it in range(ROUTIT):
            agg = edge_call(x, u, src3, dst3)
            if it < ROUTIT - 1:
                u = _norm_mid(agg, x)
            elif layer < NUM_LAYERS - 1:
                x = _norm_end(agg, x)
            else:
                out = _norm_final(agg, x)
    return out[:n]


# speedup vs baseline: 2.1197x; 1.1613x over previous
import functools

import jax
import jax.numpy as jnp
from jax import lax
from jax.experimental import pallas as pl
from jax.experimental.pallas import tpu as pltpu
from jax.experimental.pallas import tpu_sc as plsc

N_NODES = 10000
HID = 128
K = 8
D = 16
ROUTIT = 7
NUM_LAYERS = 4

NC = 2
NS = 16
WORKERS = NC * NS
B = 64
GPB = B // 16
NPAD = 10240
ROWS_PER_TILE = NPAD // NS
SCH = 64
BLK_R = 256


def _group_mat():
    r = lax.broadcasted_iota(jnp.int32, (HID, HID), 0) // D
    c = lax.broadcasted_iota(jnp.int32, (HID, HID), 1) // D
    return (r == c).astype(jnp.float32)


def _inv_norm(s):
    return jnp.minimum(lax.rsqrt(s), 1e12)


def _init_body(x_ref, w_ref, b_ref, o_ref):
    i = pl.program_id(0)
    z = jnp.dot(x_ref[...], w_ref[...], preferred_element_type=jnp.float32)
    z = jnp.maximum(z + b_ref[...], 0.0)
    row = i * BLK_R + lax.broadcasted_iota(jnp.int32, (BLK_R, HID), 0)
    z = jnp.where(row < N_NODES, z, 0.0)
    s = jnp.dot(z * z, _group_mat(), preferred_element_type=jnp.float32)
    o_ref[...] = z * _inv_norm(s)


_init_call = pl.pallas_call(
    _init_body,
    grid=(NPAD // BLK_R,),
    in_specs=[
        pl.BlockSpec((BLK_R, HID), lambda i: (i, 0)),
        pl.BlockSpec((HID, HID), lambda i: (0, 0)),
        pl.BlockSpec((1, HID), lambda i: (0, 0)),
    ],
    out_specs=pl.BlockSpec((BLK_R, HID), lambda i: (i, 0)),
    out_shape=jax.ShapeDtypeStruct((NPAD, HID), jnp.float32),
)


def _norm_body(mode, agg_ref, x_ref, o_ref):
    g = _group_mat()
    t = agg_ref[0] + agg_ref[1] + x_ref[...]
    s = jnp.dot(t * t, g, preferred_element_type=jnp.float32)
    u = t * _inv_norm(s)
    if mode == "mid":
        o_ref[...] = u
    elif mode == "final":
        o_ref[...] = jnp.maximum(u, 0.0)
    else:
        r = jnp.maximum(u, 0.0)
        s2 = jnp.dot(r * r, g, preferred_element_type=jnp.float32)
        o_ref[...] = r * _inv_norm(s2)


def _make_norm(mode):
    return pl.pallas_call(
        functools.partial(_norm_body, mode),
        grid=(NPAD // BLK_R,),
        in_specs=[
            pl.BlockSpec((NC, BLK_R, HID), lambda i: (0, i, 0)),
            pl.BlockSpec((BLK_R, HID), lambda i: (i, 0)),
        ],
        out_specs=pl.BlockSpec((BLK_R, HID), lambda i: (i, 0)),
        out_shape=jax.ShapeDtypeStruct((NPAD, HID), jnp.float32),
    )


_norm_mid = _make_norm("mid")
_norm_end = _make_norm("end")
_norm_final = _make_norm("final")


SB = 16


def _make_edge_kernel(nsb):
    mesh = plsc.VectorSubcoreMesh(core_axis_name="c", subcore_axis_name="s")

    @functools.partial(
        pl.kernel,
        out_type=jax.ShapeDtypeStruct((NC, NPAD, HID), jnp.float32),
        mesh=mesh,
        compiler_params=pltpu.CompilerParams(needs_layout_passes=False),
        scratch_types=[
            pltpu.VMEM((SB, B), jnp.int32),
            pltpu.VMEM((SB, B), jnp.int32),
            pltpu.VMEM((B, HID), jnp.float32),
            pltpu.VMEM((B, HID), jnp.float32),
            pltpu.VMEM((B, HID), jnp.float32),
            pltpu.VMEM((B, HID), jnp.float32),
            pltpu.VMEM((B, HID), jnp.float32),
            pltpu.VMEM_SHARED((NPAD, HID), jnp.float32),
            pltpu.SemaphoreType.DMA,
            pltpu.SemaphoreType.DMA,
            pltpu.SemaphoreType.DMA,
            pltpu.SemaphoreType.DMA,
            pltpu.SemaphoreType.DMA,
        ],
    )
    def edge_kernel(x_ref, u_ref, src_ref, dst_ref, out_ref,
                    srcv, dstv, z0, z1, u0, u1, mbuf, aggsh,
                    sz0, sz1, su0, su1, ssc):
        c = lax.axis_index("c")
        s = lax.axis_index("s")
        wid = c * NS + s
        base = s * ROWS_PER_TILE

        def zb(i, carry):
            for jj in range(HID // 16):
                mbuf[i, pl.ds(jj * 16, 16)] = jnp.zeros((16,), jnp.float32)
            return carry
        lax.fori_loop(0, B, zb, 0)
        for i in range(ROWS_PER_TILE // SCH):
            pltpu.sync_copy(mbuf.at[pl.ds(0, SCH)],
                            aggsh.at[pl.ds(base + i * SCH, SCH)])
        plsc.subcore_barrier()

        iota16 = lax.iota(jnp.int32, 16)
        zbufs = (z0, z1)
        ubufs = (u0, u1)
        szs = (sz0, sz1)
        sus = (su0, su1)

        def issue_gather(b, par):
            pltpu.async_copy(x_ref.at[srcv.at[b]], zbufs[par], szs[par])
            pltpu.async_copy(u_ref.at[dstv.at[b]], ubufs[par], sus[par])

        def wait_gather(b, par):
            pltpu.make_async_copy(x_ref.at[srcv.at[b]], zbufs[par], szs[par]).wait()
            pltpu.make_async_copy(u_ref.at[dstv.at[b]], ubufs[par], sus[par]).wait()

        def wait_scatter(b):
            pltpu.make_async_copy(mbuf, aggsh.at[dstv.at[b]], ssc).wait()

        kmasks = [iota16 == k for k in range(K)]
        kidx = [jnp.full((16,), k, jnp.int32) for k in range(K)]
        neg = jnp.full((16,), -1e30, jnp.float32)

        def compute_block(zb, ub):
            @plsc.parallel_loop(0, B, 1, unroll=4)
            def ebody(e):
                zrows = [zb[e, pl.ds(k * D, D)] for k in range(K)]
                pv = neg
                for k in range(K):
                    uc = ub[e, pl.ds(k * D, D)]
                    pk = jnp.sum(zrows[k] * uc)
                    pv = jnp.where(kmasks[k], pk, pv)
                m = jnp.max(pv)
                ev = jnp.exp(pv - m)
                wv = ev / jnp.sum(ev)
                for k in range(K):
                    wbk = wv.at[kidx[k]].get(mode="promise_in_bounds")
                    mbuf[e, pl.ds(k * D, D)] = zrows[k] * wbk

        def sbody(sb, carry):
            @pl.when(sb > 0)
            def _():
                wait_scatter(0)
            pltpu.sync_copy(src_ref.at[wid, pl.ds(sb * SB, SB)], srcv)
            pltpu.sync_copy(dst_ref.at[wid, pl.ds(sb * SB, SB)], dstv)
            issue_gather(0, 0)

            def pbody2(p, pcarry):
                bA = 2 * p
                issue_gather(bA + 1, 1)
                wait_gather(bA, 0)

                @pl.when(p > 0)
                def _():
                    wait_scatter(bA)
                compute_block(z0, u0)
                pltpu.async_copy(mbuf, aggsh.at[dstv.at[bA]], ssc, add=True)

                @pl.when(p < SB // 2 - 1)
                def _():
                    issue_gather(bA + 2, 0)
                wait_gather(bA + 1, 1)
                wait_scatter(bA + 1)
                compute_block(z1, u1)
                pltpu.async_copy(mbuf, aggsh.at[dstv.at[bA + 1]], ssc, add=True)
                return pcarry
            lax.fori_loop(0, SB // 2, pbody2, 0)
            return carry
        lax.fori_loop(0, nsb, sbody, 0)

        wait_scatter(0)
        plsc.subcore_barrier()
        for i in range(ROWS_PER_TILE // SCH):
            pltpu.sync_copy(aggsh.at[pl.ds(base + i * SCH, SCH)],
                            mbuf.at[pl.ds(0, SCH)])
            pltpu.sync_copy(mbuf.at[pl.ds(0, SCH)],
                            out_ref.at[c, pl.ds(base + i * SCH, SCH)])

    return edge_kernel


def kernel(X, edges, W_init, b_init):
    n, _ = X.shape
    e = edges.shape[1]
    chunk = WORKERS * B * SB
    epad = -(-e // chunk) * chunk
    nsb = epad // chunk
    nblk = nsb * SB

    Xp = jnp.pad(X, ((0, NPAD - n), (0, 0)))
    src = jnp.pad(edges[0], (0, epad - e), constant_values=NPAD - 1)
    dst = jnp.pad(edges[1], (0, epad - e), constant_values=NPAD - 1)
    src3 = src.reshape(WORKERS, nblk, B)
    dst3 = dst.reshape(WORKERS, nblk, B)

    edge_call = _make_edge_kernel(nsb)

    x = _init_call(Xp, W_init, b_init.reshape(1, HID))
    out = None
    for layer in range(NUM_LAYERS):
        u = x
        for it in range(ROUTIT):
            agg = edge_call(x, u, src3, dst3)
            if it < ROUTIT - 1:
                u = _norm_mid(agg, x)
            elif layer < NUM_LAYERS - 1:
                x = _norm_end(agg, x)
            else:
                out = _norm_final(agg, x)
    return out[:n]


# speedup vs baseline: 2.4835x; 1.1716x over previous
import functools

import jax
import jax.numpy as jnp
from jax import lax
from jax.experimental import pallas as pl
from jax.experimental.pallas import tpu as pltpu
from jax.experimental.pallas import tpu_sc as plsc

N_NODES = 10000
HID = 128
K = 8
D = 16
ROUTIT = 7
NUM_LAYERS = 4

NC = 2
NS = 16
WORKERS = NC * NS
B = 64
GPB = B // 16
NPAD = 10240
ROWS_PER_TILE = NPAD // NS
SCH = 64
BLK_R = 256


def _group_mat():
    r = lax.broadcasted_iota(jnp.int32, (HID, HID), 0) // D
    c = lax.broadcasted_iota(jnp.int32, (HID, HID), 1) // D
    return (r == c).astype(jnp.float32)


def _inv_norm(s):
    return jnp.minimum(lax.rsqrt(s), 1e12)


def _sel_mat(off):
    r = lax.broadcasted_iota(jnp.int32, (HID, HID // 2), 0)
    c = lax.broadcasted_iota(jnp.int32, (HID, HID // 2), 1)
    q = 32 * (c // D) + off + (c % D)
    return (r == q).astype(jnp.float32)


def _bf16_bits(v):
    bits = lax.bitcast_convert_type(v, jnp.int32)
    r = (bits + 0x7FFF + ((bits >> 16) & 1)) >> 16
    return r & 0xFFFF


def _bf16_pack(v):
    lo = _bf16_bits(jnp.dot(v, _sel_mat(0), preferred_element_type=jnp.float32))
    hi = _bf16_bits(jnp.dot(v, _sel_mat(D), preferred_element_type=jnp.float32))
    return lo | (hi << 16)


def _init_body(x_ref, w_ref, b_ref, o_ref, obf_ref):
    i = pl.program_id(0)
    z = jnp.dot(x_ref[...], w_ref[...], preferred_element_type=jnp.float32)
    z = jnp.maximum(z + b_ref[...], 0.0)
    row = i * BLK_R + lax.broadcasted_iota(jnp.int32, (BLK_R, HID), 0)
    z = jnp.where(row < N_NODES, z, 0.0)
    s = jnp.dot(z * z, _group_mat(), preferred_element_type=jnp.float32)
    xn = z * _inv_norm(s)
    o_ref[...] = xn
    obf_ref[...] = _bf16_pack(xn)


_init_call = pl.pallas_call(
    _init_body,
    grid=(NPAD // BLK_R,),
    in_specs=[
        pl.BlockSpec((BLK_R, HID), lambda i: (i, 0)),
        pl.BlockSpec((HID, HID), lambda i: (0, 0)),
        pl.BlockSpec((1, HID), lambda i: (0, 0)),
    ],
    out_specs=(pl.BlockSpec((BLK_R, HID), lambda i: (i, 0)),
               pl.BlockSpec((BLK_R, HID // 2), lambda i: (i, 0))),
    out_shape=(jax.ShapeDtypeStruct((NPAD, HID), jnp.float32),
               jax.ShapeDtypeStruct((NPAD, HID // 2), jnp.int32)),
)


def _norm_body(mode, agg_ref, x_ref, *o_refs):
    g = _group_mat()
    t = agg_ref[0] + agg_ref[1] + x_ref[...]
    s = jnp.dot(t * t, g, preferred_element_type=jnp.float32)
    u = t * _inv_norm(s)
    if mode == "mid":
        o_refs[0][...] = _bf16_pack(u)
    elif mode == "final":
        o_refs[0][...] = jnp.maximum(u, 0.0)
    else:
        r = jnp.maximum(u, 0.0)
        s2 = jnp.dot(r * r, g, preferred_element_type=jnp.float32)
        xn = r * _inv_norm(s2)
        o_refs[0][...] = xn
        o_refs[1][...] = _bf16_pack(xn)


def _make_norm(mode):
    blk = pl.BlockSpec((BLK_R, HID), lambda i: (i, 0))
    blkh = pl.BlockSpec((BLK_R, HID // 2), lambda i: (i, 0))
    f32sd = jax.ShapeDtypeStruct((NPAD, HID), jnp.float32)
    bfsd = jax.ShapeDtypeStruct((NPAD, HID // 2), jnp.int32)
    if mode == "mid":
        out_specs, out_shape = blkh, bfsd
    elif mode == "final":
        out_specs, out_shape = blk, f32sd
    else:
        out_specs, out_shape = (blk, blkh), (f32sd, bfsd)
    return pl.pallas_call(
        functools.partial(_norm_body, mode),
        grid=(NPAD // BLK_R,),
        in_specs=[
            pl.BlockSpec((NC, BLK_R, HID), lambda i: (0, i, 0)),
            pl.BlockSpec((BLK_R, HID), lambda i: (i, 0)),
        ],
        out_specs=out_specs,
        out_shape=out_shape,
    )


_norm_mid = _make_norm("mid")
_norm_end = _make_norm("end")
_norm_final = _make_norm("final")


SB = 16


def _make_edge_kernel(nsb):
    mesh = plsc.VectorSubcoreMesh(core_axis_name="c", subcore_axis_name="s")

    @functools.partial(
        pl.kernel,
        out_type=jax.ShapeDtypeStruct((NC, NPAD, HID), jnp.float32),
        mesh=mesh,
        compiler_params=pltpu.CompilerParams(
            needs_layout_passes=False, use_tc_tiling_on_sc=False),
        scratch_types=[
            pltpu.VMEM((SB, B), jnp.int32),
            pltpu.VMEM((SB, B), jnp.int32),
            pltpu.VMEM((B, HID // 2), jnp.int32),
            pltpu.VMEM((B, HID // 2), jnp.int32),
            pltpu.VMEM((B, HID // 2), jnp.int32),
            pltpu.VMEM((B, HID // 2), jnp.int32),
            pltpu.VMEM((B, HID), jnp.float32),
            pltpu.VMEM_SHARED((NPAD, HID), jnp.float32),
            pltpu.SemaphoreType.DMA,
            pltpu.SemaphoreType.DMA,
            pltpu.SemaphoreType.DMA,
            pltpu.SemaphoreType.DMA,
            pltpu.SemaphoreType.DMA,
        ],
    )
    def edge_kernel(x_ref, u_ref, src_ref, dst_ref, out_ref,
                    srcv, dstv, z0, z1, u0, u1, mbuf, aggsh,
                    sz0, sz1, su0, su1, ssc):
        c = lax.axis_index("c")
        s = lax.axis_index("s")
        wid = c * NS + s
        base = s * ROWS_PER_TILE

        def zb(i, carry):
            for jj in range(HID // 16):
                mbuf[i, pl.ds(jj * 16, 16)] = jnp.zeros((16,), jnp.float32)
            return carry
        lax.fori_loop(0, B, zb, 0)
        for i in range(ROWS_PER_TILE // SCH):
            pltpu.sync_copy(mbuf.at[pl.ds(0, SCH)],
                            aggsh.at[pl.ds(base + i * SCH, SCH)])
        plsc.subcore_barrier()

        iota16 = lax.iota(jnp.int32, 16)
        zbufs = (z0, z1)
        ubufs = (u0, u1)
        szs = (sz0, sz1)
        sus = (su0, su1)

        def issue_gather(b, par):
            pltpu.async_copy(x_ref.at[srcv.at[b]], zbufs[par], szs[par])
            pltpu.async_copy(u_ref.at[dstv.at[b]], ubufs[par], sus[par])

        def wait_gather(b, par):
            pltpu.make_async_copy(x_ref.at[srcv.at[b]], zbufs[par], szs[par]).wait()
            pltpu.make_async_copy(u_ref.at[dstv.at[b]], ubufs[par], sus[par]).wait()

        def wait_scatter(b):
            pltpu.make_async_copy(mbuf, aggsh.at[dstv.at[b]], ssc).wait()

        kmasks = [iota16 == k for k in range(K)]
        kidx = [jnp.full((16,), k, jnp.int32) for k in range(K)]
        neg = jnp.full((16,), -1e30, jnp.float32)

        def compute_block(zb, ub):
            @plsc.parallel_loop(0, B, 1, unroll=4)
            def ebody(e):
                zcs = []
                ucs = []
                for w in range(K // 2):
                    zw = zb[e, pl.ds(w * D, D)]
                    uw = ub[e, pl.ds(w * D, D)]
                    zcs.append(plsc.bitcast(zw << 16, jnp.float32))
                    zcs.append(plsc.bitcast(zw & -65536, jnp.float32))
                    ucs.append(plsc.bitcast(uw << 16, jnp.float32))
                    ucs.append(plsc.bitcast(uw & -65536, jnp.float32))
                pv = neg
                for k in range(K):
                    pk = jnp.sum(zcs[k] * ucs[k])
                    pv = jnp.where(kmasks[k], pk, pv)
                m = jnp.max(pv)
                ev = jnp.exp(pv - m)
                wv = ev / jnp.sum(ev)
                for k in range(K):
                    wbk = wv.at[kidx[k]].get(mode="promise_in_bounds")
                    mbuf[e, pl.ds(k * D, D)] = zcs[k] * wbk

        def sbody(sb, carry):
            @pl.when(sb > 0)
            def _():
                wait_scatter(0)
            pltpu.sync_copy(src_ref.at[wid, pl.ds(sb * SB, SB)], srcv)
            pltpu.sync_copy(dst_ref.at[wid, pl.ds(sb * SB, SB)], dstv)
            issue_gather(0, 0)

            def pbody2(p, pcarry):
                bA = 2 * p
                issue_gather(bA + 1, 1)
                wait_gather(bA, 0)

                @pl.when(p > 0)
                def _():
                    wait_scatter(bA)
                compute_block(z0, u0)
                pltpu.async_copy(mbuf, aggsh.at[dstv.at[bA]], ssc, add=True)

                @pl.when(p < SB // 2 - 1)
                def _():
                    issue_gather(bA + 2, 0)
                wait_gather(bA + 1, 1)
                wait_scatter(bA + 1)
                compute_block(z1, u1)
                pltpu.async_copy(mbuf, aggsh.at[dstv.at[bA + 1]], ssc, add=True)
                return pcarry
            lax.fori_loop(0, SB // 2, pbody2, 0)
            return carry
        lax.fori_loop(0, nsb, sbody, 0)

        wait_scatter(0)
        plsc.subcore_barrier()
        for i in range(ROWS_PER_TILE // SCH):
            pltpu.sync_copy(aggsh.at[pl.ds(base + i * SCH, SCH)],
                            mbuf.at[pl.ds(0, SCH)])
            pltpu.sync_copy(mbuf.at[pl.ds(0, SCH)],
                            out_ref.at[c, pl.ds(base + i * SCH, SCH)])

    return edge_kernel


def kernel(X, edges, W_init, b_init):
    n, _ = X.shape
    e = edges.shape[1]
    chunk = WORKERS * B * SB
    epad = -(-e // chunk) * chunk
    nsb = epad // chunk
    nblk = nsb * SB

    Xp = jnp.pad(X, ((0, NPAD - n), (0, 0)))
    src = jnp.pad(edges[0], (0, epad - e), constant_values=NPAD - 1)
    dst = jnp.pad(edges[1], (0, epad - e), constant_values=NPAD - 1)
    src3 = src.reshape(WORKERS, nblk, B)
    dst3 = dst.reshape(WORKERS, nblk, B)

    edge_call = _make_edge_kernel(nsb)

    x, xbf = _init_call(Xp, W_init, b_init.reshape(1, HID))
    out = None
    for layer in range(NUM_LAYERS):
        ubf = xbf
        for it in range(ROUTIT):
            agg = edge_call(xbf, ubf, src3, dst3)
            if it < ROUTIT - 1:
                ubf = _norm_mid(agg, x)
            elif layer < NUM_LAYERS - 1:
                x, xbf = _norm_end(agg, x)
            else:
                out = _norm_final(agg, x)
    return out[:n]
